# baseline (speedup 1.0000x reference)
import functools

import jax
import jax.numpy as jnp
from jax import lax
from jax.experimental import pallas as pl
from jax.experimental.pallas import tpu as pltpu
from jax.experimental.pallas import tpu_sc as plsc

NC = 2
NS = 16
NW = NC * NS
CHUNK = 128


@functools.partial(jax.jit, static_argnums=(2, 3))
def _sc_gather(ids3, table, nchunk, d):
    mesh = plsc.VectorSubcoreMesh(core_axis_name="c", subcore_axis_name="s")

    @functools.partial(
        pl.kernel,
        out_type=jax.ShapeDtypeStruct((NW, nchunk, CHUNK, d), jnp.float32),
        mesh=mesh,
        scratch_types=[
            pltpu.VMEM((nchunk, CHUNK), jnp.int32),
            pltpu.VMEM((CHUNK, d), jnp.float32),
            pltpu.SemaphoreType.DMA,
        ],
        compiler_params=pltpu.CompilerParams(use_tc_tiling_on_sc=False),
    )
    def k(ids_hbm, table_hbm, out_hbm, idx_v, rows_v, sem):
        wid = lax.axis_index("s") * NC + lax.axis_index("c")
        pltpu.sync_copy(ids_hbm.at[wid], idx_v)

        def body(j, carry):
            pltpu.async_copy(table_hbm.at[idx_v.at[j]], rows_v, sem).wait()
            pltpu.sync_copy(rows_v, out_hbm.at[wid].at[j])
            return carry

        lax.fori_loop(0, nchunk, body, 0)

    return k(ids3, table)


def kernel(input_ids, table):
    b, h = input_ids.shape
    v, d = table.shape
    n = b * h
    assert n % (NW * CHUNK) == 0
    nchunk = n // (NW * CHUNK)
    ids3 = input_ids.reshape(NW, nchunk, CHUNK).astype(jnp.int32)
    out = _sc_gather(ids3, table, nchunk, d)
    return out.reshape(b, h, d)[..., None, None]


# speedup vs baseline: 1.1142x; 1.1142x over previous
import functools

import jax
import jax.numpy as jnp
from jax import lax
from jax.experimental import pallas as pl
from jax.experimental.pallas import tpu as pltpu
from jax.experimental.pallas import tpu_sc as plsc

NC = 2
NS = 16
NW = NC * NS
CHUNK = 128


NBUF = 8


@functools.partial(jax.jit, static_argnums=(2, 3))
def _sc_gather(ids3, table, nchunk, d):
    mesh = plsc.VectorSubcoreMesh(core_axis_name="c", subcore_axis_name="s")
    ngroup = nchunk // NBUF
    assert ngroup * NBUF == nchunk

    @functools.partial(
        pl.kernel,
        out_type=jax.ShapeDtypeStruct((NW, nchunk, CHUNK, d), jnp.float32),
        mesh=mesh,
        scratch_types=[
            pltpu.VMEM((nchunk, CHUNK), jnp.int32),
            pltpu.VMEM((NBUF, CHUNK, d), jnp.float32),
            pltpu.SemaphoreType.DMA((NBUF,)),
            pltpu.SemaphoreType.DMA((NBUF,)),
        ],
        compiler_params=pltpu.CompilerParams(use_tc_tiling_on_sc=False),
    )
    def k(ids_hbm, table_hbm, out_hbm, idx_v, rows_v, gsem, wsem):
        wid = lax.axis_index("s") * NC + lax.axis_index("c")
        pltpu.sync_copy(ids_hbm.at[wid], idx_v)

        for b in range(NBUF):
            pltpu.async_copy(table_hbm.at[idx_v.at[b]], rows_v.at[b], gsem.at[b])

        def group(g, carry):
            for b in range(NBUF):
                j = g * NBUF + b
                pltpu.make_async_copy(
                    table_hbm.at[idx_v.at[j]], rows_v.at[b], gsem.at[b]
                ).wait()
                pltpu.async_copy(rows_v.at[b], out_hbm.at[wid].at[j], wsem.at[b])

                @pl.when(g < ngroup - 1)
                def _():
                    pltpu.make_async_copy(
                        rows_v.at[b], out_hbm.at[wid].at[j], wsem.at[b]
                    ).wait()
                    pltpu.async_copy(
                        table_hbm.at[idx_v.at[j + NBUF]], rows_v.at[b], gsem.at[b]
                    )

            return carry

        lax.fori_loop(0, ngroup, group, 0)

        for b in range(NBUF):
            j = (ngroup - 1) * NBUF + b
            pltpu.make_async_copy(
                rows_v.at[b], out_hbm.at[wid].at[j], wsem.at[b]
            ).wait()

    return k(ids3, table)


def kernel(input_ids, table):
    b, h = input_ids.shape
    v, d = table.shape
    n = b * h
    assert n % (NW * CHUNK) == 0
    nchunk = n // (NW * CHUNK)
    ids3 = input_ids.reshape(NW, nchunk, CHUNK).astype(jnp.int32)
    out = _sc_gather(ids3, table, nchunk, d)
    return out.reshape(b, h, d)[..., None, None]
